# baseline (device time: 40005 ns/iter reference)
import jax
import jax.numpy as jnp
from jax import lax
from jax.experimental import pallas as pl
from jax.experimental.pallas import tpu as pltpu

N_DEV = 4
BLK = 64


def kernel(x, Wq, K_ext, V_ext, Wo):
    B, Sq, Dm = x.shape
    _, Skv, Hq, Dh = K_ext.shape
    HD = Hq * Dh
    Dout = Wo.shape[1]

    K2 = K_ext.reshape(B, Skv, HD)
    V2 = V_ext.reshape(B, Skv, HD)

    def body(x_ref, wq_ref, k_ref, v_ref, wo_ref, out_ref,
             comm_k, comm_v, ksend, krecv, vsend, vrecv):
        my = lax.axis_index("i")
        right = (my + 1) % N_DEV
        left = (my + N_DEV - 1) % N_DEV

        barrier = pltpu.get_barrier_semaphore()
        for nbr in (left, right):
            pl.semaphore_signal(barrier, inc=1, device_id=(nbr,),
                                device_id_type=pl.DeviceIdType.MESH)
        pl.semaphore_wait(barrier, 2)

        comm_k[0] = k_ref[...].astype(jnp.bfloat16)
        comm_v[0] = v_ref[...].astype(jnp.bfloat16)

        for h in range(N_DEV - 1):
            rk = pltpu.make_async_remote_copy(
                src_ref=comm_k.at[h], dst_ref=comm_k.at[h + 1],
                send_sem=ksend.at[h], recv_sem=krecv.at[h + 1],
                device_id=(right,), device_id_type=pl.DeviceIdType.MESH)
            rv = pltpu.make_async_remote_copy(
                src_ref=comm_v.at[h], dst_ref=comm_v.at[h + 1],
                send_sem=vsend.at[h], recv_sem=vrecv.at[h + 1],
                device_id=(right,), device_id_type=pl.DeviceIdType.MESH)
            rk.start()
            rv.start()
            rk.wait()
            rv.wait()

        wq = wq_ref[...].astype(jnp.bfloat16)
        wo = wo_ref[...].astype(jnp.bfloat16)
        for b in range(B):
            q_b = jnp.dot(x_ref[b].astype(jnp.bfloat16), wq,
                          preferred_element_type=jnp.float32
                          ).astype(jnp.bfloat16)
            ctx_rows = []
            for j in range(Sq // BLK):
                rows = pl.ds(j * BLK, BLK)
                k_sel = jnp.concatenate(
                    [comm_k[t, b, rows, :] for t in range(N_DEV)], axis=0)
                v_sel = jnp.concatenate(
                    [comm_v[t, b, rows, :] for t in range(N_DEV)], axis=0)
                q_blk = q_b[j * BLK:(j + 1) * BLK, :]
                ctx_heads = []
                for hh in range(Hq):
                    cs = slice(hh * Dh, (hh + 1) * Dh)
                    scores = lax.dot_general(
                        q_blk[:, cs], k_sel[:, cs],
                        (((1,), (1,)), ((), ())),
                        preferred_element_type=jnp.float32) * 0.125
                    m = jnp.max(scores, axis=-1, keepdims=True)
                    w = jnp.exp(scores - m)
                    w = w / jnp.sum(w, axis=-1, keepdims=True)
                    ctx_heads.append(
                        jnp.dot(w.astype(jnp.bfloat16), v_sel[:, cs],
                                preferred_element_type=jnp.float32))
                ctx_rows.append(jnp.concatenate(ctx_heads, axis=1))
            ctx_b = jnp.concatenate(ctx_rows, axis=0).astype(jnp.bfloat16)
            out_ref[b] = jnp.dot(ctx_b, wo, preferred_element_type=jnp.float32)

    return pl.pallas_call(
        body,
        out_shape=jax.ShapeDtypeStruct((B, Sq, Dout), jnp.float32),
        in_specs=[pl.BlockSpec(memory_space=pltpu.VMEM)] * 5,
        out_specs=pl.BlockSpec(memory_space=pltpu.VMEM),
        scratch_shapes=[
            pltpu.VMEM((N_DEV, B, Skv, HD), jnp.bfloat16),
            pltpu.VMEM((N_DEV, B, Skv, HD), jnp.bfloat16),
            pltpu.SemaphoreType.DMA((N_DEV,)),
            pltpu.SemaphoreType.DMA((N_DEV,)),
            pltpu.SemaphoreType.DMA((N_DEV,)),
            pltpu.SemaphoreType.DMA((N_DEV,)),
        ],
        compiler_params=pltpu.CompilerParams(collective_id=0),
    )(x, Wq, K2, V2, Wo)


# device time: 19005 ns/iter; 2.1050x vs baseline; 2.1050x over previous
import functools

import jax
import jax.numpy as jnp
from jax import lax
from jax.experimental import pallas as pl
from jax.experimental.pallas import tpu as pltpu

N_DEV = 4
BLK = 64


def kernel(x, Wq, K_ext, V_ext, Wo):
    B, Sq, Dm = x.shape
    _, Skv, Hq, Dh = K_ext.shape
    HD = Hq * Dh
    Dout = Wo.shape[1]

    K2 = K_ext.reshape(B, Skv, HD)
    V2 = V_ext.reshape(B, Skv, HD)

    def body(x_ref, wq_ref, k_ref, v_ref, wo_ref, out_ref,
             comm, ssend, rrecv):
        my = lax.axis_index("i")
        peers = [(my + d) % N_DEV for d in (1, 2, 3)]

        barrier = pltpu.get_barrier_semaphore()
        for p in peers:
            pl.semaphore_signal(barrier, inc=1, device_id=(p,),
                                device_id_type=pl.DeviceIdType.MESH)
        pl.semaphore_wait(barrier, 3)

        comm[0, 0] = k_ref[...].astype(jnp.bfloat16)
        comm[0, 1] = v_ref[...].astype(jnp.bfloat16)

        rdmas = []
        for d in (1, 2, 3):
            r = pltpu.make_async_remote_copy(
                src_ref=comm.at[0], dst_ref=comm.at[N_DEV - d],
                send_sem=ssend.at[d], recv_sem=rrecv.at[N_DEV - d],
                device_id=(peers[d - 1],),
                device_id_type=pl.DeviceIdType.MESH)
            r.start()
            rdmas.append(r)

        wq = wq_ref[...].astype(jnp.bfloat16)
        wo = wo_ref[...].astype(jnp.bfloat16)
        q = [jnp.dot(x_ref[b].astype(jnp.bfloat16), wq,
                     preferred_element_type=jnp.float32
                     ).astype(jnp.bfloat16) for b in range(B)]

        for r in rdmas:
            r.wait_recv()

        for b in range(B):
            ctx_rows = []
            for j in range(Sq // BLK):
                rows = pl.ds(j * BLK, BLK)
                k_sel = jnp.concatenate(
                    [comm[t, 0, b, rows, :] for t in range(N_DEV)], axis=0)
                v_sel = jnp.concatenate(
                    [comm[t, 1, b, rows, :] for t in range(N_DEV)], axis=0)
                q_blk = q[b][j * BLK:(j + 1) * BLK, :]
                ctx_heads = []
                for hh in range(Hq):
                    cs = slice(hh * Dh, (hh + 1) * Dh)
                    scores = lax.dot_general(
                        q_blk[:, cs], k_sel[:, cs],
                        (((1,), (1,)), ((), ())),
                        preferred_element_type=jnp.float32) * 0.125
                    m = jnp.max(scores, axis=-1, keepdims=True)
                    w = jnp.exp(scores - m)
                    w = w / jnp.sum(w, axis=-1, keepdims=True)
                    ctx_heads.append(
                        jnp.dot(w.astype(jnp.bfloat16), v_sel[:, cs],
                                preferred_element_type=jnp.float32))
                ctx_rows.append(jnp.concatenate(ctx_heads, axis=1))
            ctx_b = jnp.concatenate(ctx_rows, axis=0).astype(jnp.bfloat16)
            out_ref[b] = jnp.dot(ctx_b, wo, preferred_element_type=jnp.float32)

        for r in rdmas:
            r.wait_send()

        @functools.partial(pl.run_scoped,
                           second_barrier=pltpu.SemaphoreType.REGULAR)
        def _(second_barrier):
            for p in peers:
                pl.semaphore_signal(second_barrier, inc=1, device_id=(p,),
                                    device_id_type=pl.DeviceIdType.MESH)
            pl.semaphore_wait(second_barrier, 3)

    return pl.pallas_call(
        body,
        out_shape=jax.ShapeDtypeStruct((B, Sq, Dout), jnp.float32),
        in_specs=[pl.BlockSpec(memory_space=pltpu.VMEM)] * 5,
        out_specs=pl.BlockSpec(memory_space=pltpu.VMEM),
        scratch_shapes=[
            pltpu.VMEM((N_DEV, 2, B, Skv, HD), jnp.bfloat16),
            pltpu.SemaphoreType.DMA((N_DEV,)),
            pltpu.SemaphoreType.DMA((N_DEV,)),
        ],
        compiler_params=pltpu.CompilerParams(collective_id=0),
    )(x, Wq, K2, V2, Wo)
